# Initial kernel scaffold; baseline (speedup 1.0000x reference)
import jax
import jax.numpy as jnp
from jax.experimental import pallas as pl


def kernel(x, W, b, edge_index):
    raise NotImplementedError("write your pallas kernel here")


# speedup vs baseline: 2.8298x; 2.8298x over previous
import functools

import jax
import jax.numpy as jnp
from jax import lax
from jax.experimental import pallas as pl
from jax.experimental.pallas import tpu as pltpu
from jax.experimental.pallas import tpu_sc as plsc

N = 10000
E = 160000
C_IN = 256
D = 512

NC = 2
NS = 16
L = 16

NPAD = 10240
EROWS = 1280
EPT = EROWS // NS
RPW = 160
SWEEPS = 2
ACC_TRASH = RPW
BATCH = 64
SBUF = 208
SBUF_TRASH = 192

_MESH = dict(core_axis_name="c", subcore_axis_name="s")


def _rsqrt_newton(d):
  i = lax.bitcast_convert_type(d, jnp.int32)
  i = 0x5F3759DF - lax.shift_right_arithmetic(i, jnp.full((L,), 1, jnp.int32))
  y = lax.bitcast_convert_type(i, jnp.float32)
  for _ in range(4):
    y = y * (1.5 - 0.5 * d * y * y)
  return y


def _deg_body(e_ref, dinv_ref, deg_sh, idxb, onesb, degv, dvv):
  cid = lax.axis_index("c")
  sid = lax.axis_index("s")
  w = cid * NS + sid

  for k in range(128 // L):
    onesb[0, pl.ds(k * L, L)] = jnp.full((L,), 1.0, jnp.float32)
  for blk in range(NPAD // NS // 128):
    pltpu.sync_copy(
        onesb.at[0], deg_sh.at[pl.ds(sid * (NPAD // NS) + blk * 128, 128)])
  plsc.subcore_barrier()

  def _hist_rows(r0, nr):
    pltpu.sync_copy(e_ref.at[1, pl.ds(r0, nr), :], idxb.at[pl.ds(0, nr), :])
    for j in range(nr):
      pltpu.sync_copy(onesb.at[0], deg_sh.at[idxb.at[j]], add=True)

  for blk in range(EPT // 16):
    _hist_rows(sid * EPT + blk * 16, 16)

  plsc.subcore_barrier()

  pltpu.sync_copy(deg_sh.at[pl.ds(w * 320, 320)], degv)
  for k in range(320 // L):
    dvv[pl.ds(k * L, L)] = _rsqrt_newton(degv[pl.ds(k * L, L)])
  pltpu.sync_copy(dvv, dinv_ref.at[pl.ds(w * 320, 320)])


@functools.partial(
    pl.kernel,
    out_type=jax.ShapeDtypeStruct((NPAD,), jnp.float32),
    mesh=plsc.VectorSubcoreMesh(**_MESH),
    scratch_types=[
        pltpu.VMEM_SHARED((NPAD,), jnp.float32),
        pltpu.VMEM((16, 128), jnp.int32),
        pltpu.VMEM((1, 128), jnp.float32),
        pltpu.VMEM((320,), jnp.float32),
        pltpu.VMEM((320,), jnp.float32),
    ],
)
def _deg_kernel(e_ref, dinv_ref, deg_sh, idxb, onesb, degv, dvv):
  _deg_body(e_ref, dinv_ref, deg_sh, idxb, onesb, degv, dvv)


def _mm_body(x_ref, w_ref, dv_ref, g_ref):
  g_ref[...] = dv_ref[...] * jnp.dot(
      x_ref[...], w_ref[...], preferred_element_type=jnp.float32)


def _matmul(x, W, dinv2):
  blk = 400
  return pl.pallas_call(
      _mm_body,
      grid=(N // blk,),
      in_specs=[
          pl.BlockSpec((blk, C_IN), lambda i: (i, 0)),
          pl.BlockSpec((C_IN, D), lambda i: (0, 0)),
          pl.BlockSpec((blk, 1), lambda i: (i, 0)),
      ],
      out_specs=pl.BlockSpec((blk, D), lambda i: (i, 0)),
      out_shape=jax.ShapeDtypeStruct((N, D), jnp.float32),
  )(x, W, dinv2)


def _agg_body(e_ref, g_ref, dinv_ref, b_ref, out_ref,
              acc, srcb, dstb, sbg, sbs, gidx, rows_v, bb, dvv):
  cid = lax.axis_index("c")
  sid = lax.axis_index("s")
  w = cid * NS + sid

  pltpu.sync_copy(b_ref, bb)

  def _accumulate(cnt):
    cl = jnp.minimum(cnt, BATCH)
    for k in range(BATCH // L):
      lane = lax.iota(jnp.int32, L) + k * L
      sel = lane < cl
      gidx[0, pl.ds(k * L, L)] = jnp.where(sel, sbg[pl.ds(k * L, L)], 0)
    pltpu.sync_copy(g_ref.at[gidx.at[0]], rows_v)

    def _grp(grp, _):
      lane = lax.iota(jnp.int32, L) + grp * L
      dstl = jnp.where(lane < cl, sbs[pl.ds(grp * L, L)], ACC_TRASH)
      dls = [dstl[j] for j in range(L)]

      def _add(cc, _):
        off = cc * L
        for j in range(L):
          r = grp * L + j
          acc[dls[j], pl.ds(off, L)] = (
              acc[dls[j], pl.ds(off, L)] + rows_v[r, pl.ds(off, L)])
        return 0

      lax.fori_loop(0, D // L, _add, 0)
      return 0

    lax.fori_loop(0, BATCH // L, _grp, 0)

  def _sweep(s, _):
    gbase = (s * (NC * NS) + w) * RPW
    n16 = jnp.clip((N - gbase) // 16, 0, RPW // 16)

    def _init(i, _):
      pltpu.sync_copy(g_ref.at[pl.ds(gbase + i * 16, 16), :],
                      acc.at[pl.ds(i * 16, 16), :])
      return 0

    lax.fori_loop(0, n16, _init, 0)

    def _scan_blk(b, cnt):
      pltpu.sync_copy(e_ref.at[0, pl.ds(b * 16, 16), :], srcb)
      pltpu.sync_copy(e_ref.at[1, pl.ds(b * 16, 16), :], dstb)

      def _row(j, cnt):
        for k in range(8):
          s16 = srcb[j, pl.ds(k * L, L)]
          d16 = dstb[j, pl.ds(k * L, L)]
          m = (d16 >= gbase) & (d16 < gbase + RPW)
          mi = m.astype(jnp.int32)
          csum = jnp.cumsum(mi)
          pos = jnp.where(m, cnt + csum - mi, SBUF_TRASH)
          plsc.store_scatter(sbg, [pos], s16)
          plsc.store_scatter(sbs, [pos], d16 - gbase)
          cnt = cnt + csum[L - 1]

        def _drain(cnt):
          _accumulate(cnt)
          for k in range(8):
            sbg[pl.ds(k * L, L)] = sbg[pl.ds(BATCH + k * L, L)]
            sbs[pl.ds(k * L, L)] = sbs[pl.ds(BATCH + k * L, L)]
          return cnt - BATCH

        return lax.while_loop(lambda c: c >= BATCH, _drain, cnt)

      return lax.fori_loop(0, 16, _row, cnt)

    cnt = lax.fori_loop(0, EROWS // 16, _scan_blk, jnp.int32(0))

    @pl.when(cnt > 0)
    def _flush():
      _accumulate(cnt)

    def _fin(i, _):
      grow = gbase + i * 16
      pltpu.sync_copy(dinv_ref.at[pl.ds(grow, 16)], dvv)
      dvec = dvv[...]
      djs = [dvec[j] for j in range(16)]

      def _scale(cc, _):
        off = cc * L
        bchunk = bb[pl.ds(off, L)]
        for j in range(16):
          r = i * 16 + j
          acc[r, pl.ds(off, L)] = djs[j] * acc[r, pl.ds(off, L)] + bchunk
        return 0

      lax.fori_loop(0, D // L, _scale, 0)
      pltpu.sync_copy(acc.at[pl.ds(i * 16, 16), :],
                      out_ref.at[pl.ds(grow, 16), :])
      return 0

    lax.fori_loop(0, n16, _fin, 0)
    return 0

  lax.fori_loop(0, SWEEPS, _sweep, 0)


@functools.partial(
    pl.kernel,
    out_type=jax.ShapeDtypeStruct((N, D), jnp.float32),
    mesh=plsc.VectorSubcoreMesh(**_MESH),
    compiler_params=pltpu.CompilerParams(needs_layout_passes=False),
    scratch_types=[
        pltpu.VMEM((RPW + 1, D), jnp.float32),
        pltpu.VMEM((16, 128), jnp.int32),
        pltpu.VMEM((16, 128), jnp.int32),
        pltpu.VMEM((SBUF,), jnp.int32),
        pltpu.VMEM((SBUF,), jnp.int32),
        pltpu.VMEM((1, BATCH), jnp.int32),
        pltpu.VMEM((BATCH, D), jnp.float32),
        pltpu.VMEM((D,), jnp.float32),
        pltpu.VMEM((16,), jnp.float32),
    ],
)
def _agg_kernel(e_ref, g_ref, dinv_ref, b_ref, out_ref,
                acc, srcb, dstb, sbg, sbs, gidx, rows_v, bb, dvv):
  _agg_body(e_ref, g_ref, dinv_ref, b_ref, out_ref,
            acc, srcb, dstb, sbg, sbs, gidx, rows_v, bb, dvv)


def kernel(x, W, b, edge_index):
  npad_e = EROWS * 128 - E
  pad = jnp.stack([
      jnp.zeros((npad_e,), edge_index.dtype),
      jnp.full((npad_e,), NPAD - 1, edge_index.dtype),
  ])
  e3 = jnp.concatenate([edge_index, pad], axis=1).reshape(2, EROWS, 128)
  dinv = _deg_kernel(e3)
  g = _matmul(x, W, dinv[:N].reshape(N, 1))
  return _agg_kernel(e3, g, dinv, b)


# speedup vs baseline: 2.9049x; 1.0266x over previous
import functools

import jax
import jax.numpy as jnp
from jax import lax
from jax.experimental import pallas as pl
from jax.experimental.pallas import tpu as pltpu
from jax.experimental.pallas import tpu_sc as plsc

N = 10000
E = 160000
C_IN = 256
D = 512

NC = 2
NS = 16
L = 16

NPAD = 10240
EROWS = 1280
EPT = EROWS // NS
RPW = 160
SWEEPS = 2
ACC_TRASH = RPW
BATCH = 64
SBUF = 208
SBUF_TRASH = 192
TBUF = 272
TBUF_TRASH = 256
NW = NC * NS

_MESH = dict(core_axis_name="c", subcore_axis_name="s")


def _rsqrt_newton(d):
  i = lax.bitcast_convert_type(d, jnp.int32)
  i = 0x5F3759DF - lax.shift_right_arithmetic(i, jnp.full((L,), 1, jnp.int32))
  y = lax.bitcast_convert_type(i, jnp.float32)
  for _ in range(4):
    y = y * (1.5 - 0.5 * d * y * y)
  return y


def _deg_body(e_ref, dinv_ref, deg_sh, idxb, onesb, degv, dvv):
  cid = lax.axis_index("c")
  sid = lax.axis_index("s")
  w = cid * NS + sid

  for k in range(128 // L):
    onesb[0, pl.ds(k * L, L)] = jnp.full((L,), 1.0, jnp.float32)
  for blk in range(NPAD // NS // 128):
    pltpu.sync_copy(
        onesb.at[0], deg_sh.at[pl.ds(sid * (NPAD // NS) + blk * 128, 128)])
  plsc.subcore_barrier()

  def _hist_rows(r0, nr):
    pltpu.sync_copy(e_ref.at[1, pl.ds(r0, nr), :], idxb.at[pl.ds(0, nr), :])
    for j in range(nr):
      pltpu.sync_copy(onesb.at[0], deg_sh.at[idxb.at[j]], add=True)

  for blk in range(EPT // 16):
    _hist_rows(sid * EPT + blk * 16, 16)

  plsc.subcore_barrier()

  pltpu.sync_copy(deg_sh.at[pl.ds(w * 320, 320)], degv)
  for k in range(320 // L):
    dvv[pl.ds(k * L, L)] = _rsqrt_newton(degv[pl.ds(k * L, L)])
  pltpu.sync_copy(dvv, dinv_ref.at[pl.ds(w * 320, 320)])


@functools.partial(
    pl.kernel,
    out_type=jax.ShapeDtypeStruct((NPAD,), jnp.float32),
    mesh=plsc.VectorSubcoreMesh(**_MESH),
    scratch_types=[
        pltpu.VMEM_SHARED((NPAD,), jnp.float32),
        pltpu.VMEM((16, 128), jnp.int32),
        pltpu.VMEM((1, 128), jnp.float32),
        pltpu.VMEM((320,), jnp.float32),
        pltpu.VMEM((320,), jnp.float32),
    ],
)
def _deg_kernel(e_ref, dinv_ref, deg_sh, idxb, onesb, degv, dvv):
  _deg_body(e_ref, dinv_ref, deg_sh, idxb, onesb, degv, dvv)


def _mm_body(x_ref, w_ref, dv_ref, g_ref):
  g_ref[...] = dv_ref[...] * jnp.dot(
      x_ref[...], w_ref[...], preferred_element_type=jnp.float32)


def _matmul(x, W, dinv2):
  blk = 400
  return pl.pallas_call(
      _mm_body,
      grid=(N // blk,),
      in_specs=[
          pl.BlockSpec((blk, C_IN), lambda i: (i, 0)),
          pl.BlockSpec((C_IN, D), lambda i: (0, 0)),
          pl.BlockSpec((blk, 1), lambda i: (i, 0)),
      ],
      out_specs=pl.BlockSpec((blk, D), lambda i: (i, 0)),
      out_shape=jax.ShapeDtypeStruct((N, D), jnp.float32),
  )(x, W, dinv2)


def _agg_body(e_ref, g_ref, dinv_ref, b_ref, out_ref,
              acc, srcb, dstb, sbg, sbs, tbg, tbs, gidx, rows_v, bb, dvv,
              rbg, rbs, sp_s, sp_d):
  cid = lax.axis_index("c")
  sid = lax.axis_index("s")
  w = cid * NS + sid
  gbase0 = w * RPW
  gbase1 = (NC * NS + w) * RPW

  pltpu.sync_copy(b_ref, bb)

  def _accumulate(cnt, off0, src_g, src_d):
    cl = jnp.minimum(cnt, BATCH)
    for k in range(BATCH // L):
      lane = lax.iota(jnp.int32, L) + k * L
      sel = lane < cl
      gidx[0, pl.ds(k * L, L)] = jnp.where(
          sel, src_g[pl.ds(off0 + k * L, L)], 0)
    pltpu.sync_copy(g_ref.at[gidx.at[0]], rows_v)

    def _grp(grp, _):
      lane = lax.iota(jnp.int32, L) + grp * L
      dstl = jnp.where(
          lane < cl, src_d[pl.ds(off0 + grp * L, L)], ACC_TRASH)
      dls = [dstl[j] for j in range(L)]

      def _add(cc, _):
        off = cc * L
        for j in range(L):
          r = grp * L + j
          acc[dls[j], pl.ds(off, L)] = (
              acc[dls[j], pl.ds(off, L)] + rows_v[r, pl.ds(off, L)])
        return 0

      lax.fori_loop(0, D // L, _add, 0)
      return 0

    lax.fori_loop(0, BATCH // L, _grp, 0)

  def _init_acc(gbase, n16):
    def _init(i, _):
      pltpu.sync_copy(g_ref.at[pl.ds(gbase + i * 16, 16), :],
                      acc.at[pl.ds(i * 16, 16), :])
      return 0

    lax.fori_loop(0, n16, _init, 0)

  def _finalize(gbase, n16):
    def _fin(i, _):
      grow = gbase + i * 16
      pltpu.sync_copy(dinv_ref.at[pl.ds(grow, 16)], dvv)
      dvec = dvv[...]
      djs = [dvec[j] for j in range(16)]

      def _scale(cc, _):
        off = cc * L
        bchunk = bb[pl.ds(off, L)]
        for j in range(16):
          r = i * 16 + j
          acc[r, pl.ds(off, L)] = djs[j] * acc[r, pl.ds(off, L)] + bchunk
        return 0

      lax.fori_loop(0, D // L, _scale, 0)
      pltpu.sync_copy(acc.at[pl.ds(i * 16, 16), :],
                      out_ref.at[pl.ds(grow, 16), :])
      return 0

    lax.fori_loop(0, n16, _fin, 0)

  n16_0 = jnp.clip((N - gbase0) // 16, 0, RPW // 16)
  n16_1 = jnp.clip((N - gbase1) // 16, 0, RPW // 16)

  _init_acc(gbase0, n16_0)

  def _scan_blk(b, carry):
    cnt0, cnt1, srow = carry
    pltpu.sync_copy(e_ref.at[0, pl.ds(b * 16, 16), :], srcb)
    pltpu.sync_copy(e_ref.at[1, pl.ds(b * 16, 16), :], dstb)

    def _row(j, carry):
      cnt0, cnt1, srow = carry
      for k in range(8):
        d16 = dstb[j, pl.ds(k * L, L)]
        m0 = (d16 >= gbase0) & (d16 < gbase0 + RPW)
        m1 = (d16 >= gbase1) & (d16 < gbase1 + RPW)
        pc0 = plsc.all_reduce_population_count(m0)[0]
        pc1 = plsc.all_reduce_population_count(m1)[0]

        @pl.when(pc0 > 0)
        def _store0(cnt0=cnt0, m0=m0, d16=d16, j=j, k=k):
          s16 = srcb[j, pl.ds(k * L, L)]
          mi = m0.astype(jnp.int32)
          csum = jnp.cumsum(mi)
          pos = jnp.where(m0, cnt0 + csum - mi, SBUF_TRASH)
          plsc.store_scatter(sbg, [pos], s16)
          plsc.store_scatter(sbs, [pos], d16 - gbase0)

        @pl.when(pc1 > 0)
        def _store1(cnt1=cnt1, m1=m1, d16=d16, j=j, k=k):
          s16 = srcb[j, pl.ds(k * L, L)]
          mi = m1.astype(jnp.int32)
          csum = jnp.cumsum(mi)
          pos = jnp.where(m1, cnt1 + csum - mi, TBUF_TRASH)
          plsc.store_scatter(tbg, [pos], s16)
          plsc.store_scatter(tbs, [pos], d16 - gbase1)

        cnt0 = cnt0 + pc0
        cnt1 = cnt1 + pc1

      def _drain0(c):
        _accumulate(c, 0, sbg, sbs)
        for k in range(8):
          sbg[pl.ds(k * L, L)] = sbg[pl.ds(BATCH + k * L, L)]
          sbs[pl.ds(k * L, L)] = sbs[pl.ds(BATCH + k * L, L)]
        return c - BATCH

      cnt0 = lax.while_loop(lambda c: c >= BATCH, _drain0, cnt0)

      def _drain1(cs):
        c, r = cs
        soff = w * (EROWS * 128) + r * 128
        pltpu.sync_copy(tbg.at[pl.ds(0, 128)], sp_s.at[pl.ds(soff, 128)])
        pltpu.sync_copy(tbs.at[pl.ds(0, 128)], sp_d.at[pl.ds(soff, 128)])
        for k in range(8):
          tbg[pl.ds(k * L, L)] = tbg[pl.ds(128 + k * L, L)]
          tbs[pl.ds(k * L, L)] = tbs[pl.ds(128 + k * L, L)]
        return c - 128, r + 1

      cnt1, srow = lax.while_loop(lambda cs: cs[0] >= 128, _drain1,
                                  (cnt1, srow))
      return cnt0, cnt1, srow

    return lax.fori_loop(0, 16, _row, (cnt0, cnt1, srow))

  cnt0, cnt1, srow = lax.fori_loop(
      0, EROWS // 16, _scan_blk,
      (jnp.int32(0), jnp.int32(0), jnp.int32(0)))

  @pl.when(cnt0 > 0)
  def _flush0():
    _accumulate(cnt0, 0, sbg, sbs)

  @pl.when(cnt1 > 0)
  def _flush1():
    for k in range(8):
      lane = lax.iota(jnp.int32, L) + k * L
      sel = lane < cnt1
      tbg[pl.ds(k * L, L)] = jnp.where(sel, tbg[pl.ds(k * L, L)], 0)
      tbs[pl.ds(k * L, L)] = jnp.where(sel, tbs[pl.ds(k * L, L)], ACC_TRASH)
    soff = w * (EROWS * 128) + srow * 128
    pltpu.sync_copy(tbg.at[pl.ds(0, 128)], sp_s.at[pl.ds(soff, 128)])
    pltpu.sync_copy(tbs.at[pl.ds(0, 128)], sp_d.at[pl.ds(soff, 128)])

  nsp = srow + jnp.where(cnt1 > 0, 1, 0)
  _finalize(gbase0, n16_0)

  _init_acc(gbase1, n16_1)
  wbase = w * (EROWS * 128)
  nblk = nsp // 16
  nrem = nsp - nblk * 16

  def _replay_blk(ib, _):
    pltpu.sync_copy(sp_s.at[pl.ds(wbase + ib * 2048, 2048)], rbg)
    pltpu.sync_copy(sp_d.at[pl.ds(wbase + ib * 2048, 2048)], rbs)

    def _rbatch(j, _):
      _accumulate(jnp.int32(BATCH), j * BATCH, rbg, rbs)
      return 0

    lax.fori_loop(0, 2048 // BATCH, _rbatch, 0)
    return 0

  lax.fori_loop(0, nblk, _replay_blk, 0)

  def _replay_row(ir, _):
    roff = wbase + nblk * 2048 + ir * 128
    pltpu.sync_copy(sp_s.at[pl.ds(roff, 128)], rbg.at[pl.ds(0, 128)])
    pltpu.sync_copy(sp_d.at[pl.ds(roff, 128)], rbs.at[pl.ds(0, 128)])
    _accumulate(jnp.int32(BATCH), 0, rbg, rbs)
    _accumulate(jnp.int32(BATCH), BATCH, rbg, rbs)
    return 0

  lax.fori_loop(0, nrem, _replay_row, 0)
  _finalize(gbase1, n16_1)


@functools.partial(
    pl.kernel,
    out_type=jax.ShapeDtypeStruct((N, D), jnp.float32),
    mesh=plsc.VectorSubcoreMesh(**_MESH),
    compiler_params=pltpu.CompilerParams(needs_layout_passes=False),
    scratch_types=[
        pltpu.VMEM((RPW + 1, D), jnp.float32),
        pltpu.VMEM((16, 128), jnp.int32),
        pltpu.VMEM((16, 128), jnp.int32),
        pltpu.VMEM((SBUF,), jnp.int32),
        pltpu.VMEM((SBUF,), jnp.int32),
        pltpu.VMEM((TBUF,), jnp.int32),
        pltpu.VMEM((TBUF,), jnp.int32),
        pltpu.VMEM((1, BATCH), jnp.int32),
        pltpu.VMEM((BATCH, D), jnp.float32),
        pltpu.VMEM((D,), jnp.float32),
        pltpu.VMEM((16,), jnp.float32),
        pltpu.VMEM((2048,), jnp.int32),
        pltpu.VMEM((2048,), jnp.int32),
        pltpu.MemorySpace.HBM((NW * EROWS * 128,), jnp.int32),
        pltpu.MemorySpace.HBM((NW * EROWS * 128,), jnp.int32),
    ],
)
def _agg_kernel(e_ref, g_ref, dinv_ref, b_ref, out_ref,
                acc, srcb, dstb, sbg, sbs, tbg, tbs, gidx, rows_v, bb, dvv,
                rbg, rbs, sp_s, sp_d):
  _agg_body(e_ref, g_ref, dinv_ref, b_ref, out_ref,
            acc, srcb, dstb, sbg, sbs, tbg, tbs, gidx, rows_v, bb, dvv,
            rbg, rbs, sp_s, sp_d)


def kernel(x, W, b, edge_index):
  npad_e = EROWS * 128 - E
  pad = jnp.stack([
      jnp.zeros((npad_e,), edge_index.dtype),
      jnp.full((npad_e,), NPAD - 1, edge_index.dtype),
  ])
  e3 = jnp.concatenate([edge_index, pad], axis=1).reshape(2, EROWS, 128)
  dinv = _deg_kernel(e3)
  g = _matmul(x, W, dinv[:N].reshape(N, 1))
  return _agg_kernel(e3, g, dinv, b)


# speedup vs baseline: 3.1585x; 1.0873x over previous
import functools

import jax
import jax.numpy as jnp
from jax import lax
from jax.experimental import pallas as pl
from jax.experimental.pallas import tpu as pltpu
from jax.experimental.pallas import tpu_sc as plsc

N = 10000
E = 160000
C_IN = 256
D = 512

NC = 2
NS = 16
L = 16

NPAD = 10240
EROWS = 1280
EPT = EROWS // NS
RPW = 160
SWEEPS = 2
ACC_TRASH = RPW
BATCH = 64
SBUF = 208
SBUF_TRASH = 192
TBUF = 272
TBUF_TRASH = 256
NW = NC * NS

_MESH = dict(core_axis_name="c", subcore_axis_name="s")


def _rsqrt_newton(d):
  i = lax.bitcast_convert_type(d, jnp.int32)
  i = 0x5F3759DF - lax.shift_right_arithmetic(i, jnp.full((L,), 1, jnp.int32))
  y = lax.bitcast_convert_type(i, jnp.float32)
  for _ in range(4):
    y = y * (1.5 - 0.5 * d * y * y)
  return y


def _deg_body(e_ref, dinv_ref, deg_sh, idxb, onesb, degv, dvv):
  cid = lax.axis_index("c")
  sid = lax.axis_index("s")
  w = cid * NS + sid

  for k in range(128 // L):
    onesb[0, pl.ds(k * L, L)] = jnp.full((L,), 1.0, jnp.float32)
  for blk in range(NPAD // NS // 128):
    pltpu.sync_copy(
        onesb.at[0], deg_sh.at[pl.ds(sid * (NPAD // NS) + blk * 128, 128)])
  plsc.subcore_barrier()

  def _hist_rows(r0, nr):
    pltpu.sync_copy(e_ref.at[1, pl.ds(r0, nr), :], idxb.at[pl.ds(0, nr), :])
    for j in range(nr):
      pltpu.sync_copy(onesb.at[0], deg_sh.at[idxb.at[j]], add=True)

  for blk in range(EPT // 16):
    _hist_rows(sid * EPT + blk * 16, 16)

  plsc.subcore_barrier()

  pltpu.sync_copy(deg_sh.at[pl.ds(w * 320, 320)], degv)
  for k in range(320 // L):
    dvv[pl.ds(k * L, L)] = _rsqrt_newton(degv[pl.ds(k * L, L)])
  pltpu.sync_copy(dvv, dinv_ref.at[pl.ds(w * 320, 320)])


@functools.partial(
    pl.kernel,
    out_type=jax.ShapeDtypeStruct((NPAD,), jnp.float32),
    mesh=plsc.VectorSubcoreMesh(**_MESH),
    scratch_types=[
        pltpu.VMEM_SHARED((NPAD,), jnp.float32),
        pltpu.VMEM((16, 128), jnp.int32),
        pltpu.VMEM((1, 128), jnp.float32),
        pltpu.VMEM((320,), jnp.float32),
        pltpu.VMEM((320,), jnp.float32),
    ],
)
def _deg_kernel(e_ref, dinv_ref, deg_sh, idxb, onesb, degv, dvv):
  _deg_body(e_ref, dinv_ref, deg_sh, idxb, onesb, degv, dvv)


def _mm_body(x_ref, w_ref, dv_ref, g_ref):
  g_ref[...] = dv_ref[...] * jnp.dot(
      x_ref[...], w_ref[...], preferred_element_type=jnp.float32)


def _matmul(x, W, dinv2):
  blk = 400
  return pl.pallas_call(
      _mm_body,
      grid=(N // blk,),
      in_specs=[
          pl.BlockSpec((blk, C_IN), lambda i: (i, 0)),
          pl.BlockSpec((C_IN, D), lambda i: (0, 0)),
          pl.BlockSpec((blk, 1), lambda i: (i, 0)),
      ],
      out_specs=pl.BlockSpec((blk, D), lambda i: (i, 0)),
      out_shape=jax.ShapeDtypeStruct((N, D), jnp.float32),
  )(x, W, dinv2)


def _agg_body(e_ref, g_ref, dinv_ref, b_ref, out_ref,
              acc, srcb, dstb, sbg, sbs, tbg, tbs, gidx, rows_v, bb, dvv,
              rbg, rbs, srcb2, dstb2, pdl, gsem, esem0, esem1, sp_s, sp_d):
  cid = lax.axis_index("c")
  sid = lax.axis_index("s")
  w = cid * NS + sid
  gbase0 = w * RPW
  gbase1 = (NC * NS + w) * RPW

  pltpu.sync_copy(b_ref, bb)

  def _snapshot(cnt, off0, src_g, src_d):
    cl = jnp.minimum(cnt, BATCH)
    for k in range(BATCH // L):
      lane = lax.iota(jnp.int32, L) + k * L
      sel = lane < cl
      gidx[0, pl.ds(k * L, L)] = jnp.where(
          sel, src_g[pl.ds(off0 + k * L, L)], 0)
      pdl[pl.ds(k * L, L)] = jnp.where(
          sel, src_d[pl.ds(off0 + k * L, L)], ACC_TRASH)

  def _gfire():
    pltpu.async_copy(g_ref.at[gidx.at[0]], rows_v, gsem)

  def _gwait():
    pltpu.make_async_copy(g_ref.at[gidx.at[0]], rows_v, gsem).wait()

  def _acc_pend():
    def _grp(grp, _):
      dstl = pdl[pl.ds(grp * L, L)]
      dls = [dstl[j] for j in range(L)]

      def _add(cc, _):
        off = cc * L
        for j in range(L):
          r = grp * L + j
          acc[dls[j], pl.ds(off, L)] = (
              acc[dls[j], pl.ds(off, L)] + rows_v[r, pl.ds(off, L)])
        return 0

      lax.fori_loop(0, D // L, _add, 0)
      return 0

    lax.fori_loop(0, BATCH // L, _grp, 0)

  def _sync_batch(cnt, off0, src_g, src_d):
    _snapshot(cnt, off0, src_g, src_d)
    _gfire()
    _gwait()
    _acc_pend()

  def _init_acc(gbase, n16):
    def _init(i, _):
      pltpu.sync_copy(g_ref.at[pl.ds(gbase + i * 16, 16), :],
                      acc.at[pl.ds(i * 16, 16), :])
      return 0

    lax.fori_loop(0, n16, _init, 0)

  def _finalize(gbase, n16):
    def _fin(i, _):
      grow = gbase + i * 16
      pltpu.sync_copy(dinv_ref.at[pl.ds(grow, 16)], dvv)
      dvec = dvv[...]
      djs = [dvec[j] for j in range(16)]

      def _scale(cc, _):
        off = cc * L
        bchunk = bb[pl.ds(off, L)]
        for j in range(16):
          r = i * 16 + j
          acc[r, pl.ds(off, L)] = djs[j] * acc[r, pl.ds(off, L)] + bchunk
        return 0

      lax.fori_loop(0, D // L, _scale, 0)
      pltpu.sync_copy(acc.at[pl.ds(i * 16, 16), :],
                      out_ref.at[pl.ds(grow, 16), :])
      return 0

    lax.fori_loop(0, n16, _fin, 0)

  n16_0 = jnp.clip((N - gbase0) // 16, 0, RPW // 16)
  n16_1 = jnp.clip((N - gbase1) // 16, 0, RPW // 16)

  _init_acc(gbase0, n16_0)

  def _fire_blk(b, sb, db, sem):
    pltpu.async_copy(e_ref.at[0, pl.ds(b * 16, 16), :], sb, sem)
    pltpu.async_copy(e_ref.at[1, pl.ds(b * 16, 16), :], db, sem)

  def _wait_blk(sb, db, sem):
    pltpu.make_async_copy(e_ref.at[0, pl.ds(0, 16), :], sb, sem).wait()
    pltpu.make_async_copy(e_ref.at[1, pl.ds(0, 16), :], db, sem).wait()

  def _scan_rows(srcb, dstb, carry):
    def _row(j, carry):
      cnt0, cnt1, srow, pend = carry
      for k in range(8):
        d16 = dstb[j, pl.ds(k * L, L)]
        m0 = (d16 >= gbase0) & (d16 < gbase0 + RPW)
        m1 = (d16 >= gbase1) & (d16 < gbase1 + RPW)
        pc0 = plsc.all_reduce_population_count(m0)[0]
        pc1 = plsc.all_reduce_population_count(m1)[0]

        @pl.when(pc0 > 0)
        def _store0(cnt0=cnt0, m0=m0, d16=d16, j=j, k=k):
          s16 = srcb[j, pl.ds(k * L, L)]
          mi = m0.astype(jnp.int32)
          csum = jnp.cumsum(mi)
          pos = jnp.where(m0, cnt0 + csum - mi, SBUF_TRASH)
          plsc.store_scatter(sbg, [pos], s16)
          plsc.store_scatter(sbs, [pos], d16 - gbase0)

        @pl.when(pc1 > 0)
        def _store1(cnt1=cnt1, m1=m1, d16=d16, j=j, k=k):
          s16 = srcb[j, pl.ds(k * L, L)]
          mi = m1.astype(jnp.int32)
          csum = jnp.cumsum(mi)
          pos = jnp.where(m1, cnt1 + csum - mi, TBUF_TRASH)
          plsc.store_scatter(tbg, [pos], s16)
          plsc.store_scatter(tbs, [pos], d16 - gbase1)

        cnt0 = cnt0 + pc0
        cnt1 = cnt1 + pc1

      def _drain0(cp):
        c, pend = cp

        @pl.when(pend == 1)
        def _settle():
          _gwait()
          _acc_pend()

        _snapshot(c, 0, sbg, sbs)
        _gfire()
        for k in range(8):
          sbg[pl.ds(k * L, L)] = sbg[pl.ds(BATCH + k * L, L)]
          sbs[pl.ds(k * L, L)] = sbs[pl.ds(BATCH + k * L, L)]
        return c - BATCH, jnp.int32(1)

      cnt0, pend = lax.while_loop(lambda cp: cp[0] >= BATCH, _drain0,
                                  (cnt0, pend))

      def _drain1(cs):
        c, r = cs
        soff = w * (EROWS * 128) + r * 128
        pltpu.sync_copy(tbg.at[pl.ds(0, 128)], sp_s.at[pl.ds(soff, 128)])
        pltpu.sync_copy(tbs.at[pl.ds(0, 128)], sp_d.at[pl.ds(soff, 128)])
        for k in range(8):
          tbg[pl.ds(k * L, L)] = tbg[pl.ds(128 + k * L, L)]
          tbs[pl.ds(k * L, L)] = tbs[pl.ds(128 + k * L, L)]
        return c - 128, r + 1

      cnt1, srow = lax.while_loop(lambda cs: cs[0] >= 128, _drain1,
                                  (cnt1, srow))
      return cnt0, cnt1, srow, pend

    return lax.fori_loop(0, 16, _row, carry)

  nblks = EROWS // 16
  _fire_blk(0, srcb, dstb, esem0)

  def _scan_pair(i, carry):
    for p, (sb, db, sem), (osb, odb, osem) in (
        (0, (srcb, dstb, esem0), (srcb2, dstb2, esem1)),
        (1, (srcb2, dstb2, esem1), (srcb, dstb, esem0)),
    ):
      b = i * 2 + p
      _fire_blk(jnp.minimum(b + 1, nblks - 1), osb, odb, osem)
      _wait_blk(sb, db, sem)
      carry = _scan_rows(sb, db, carry)
    return carry

  cnt0, cnt1, srow, pend = lax.fori_loop(
      0, nblks // 2, _scan_pair,
      (jnp.int32(0), jnp.int32(0), jnp.int32(0), jnp.int32(0)))
  _wait_blk(srcb, dstb, esem0)

  @pl.when(pend == 1)
  def _settle_tail():
    _gwait()
    _acc_pend()

  @pl.when(cnt0 > 0)
  def _flush0():
    _sync_batch(cnt0, 0, sbg, sbs)

  @pl.when(cnt1 > 0)
  def _flush1():
    for k in range(8):
      lane = lax.iota(jnp.int32, L) + k * L
      sel = lane < cnt1
      tbg[pl.ds(k * L, L)] = jnp.where(sel, tbg[pl.ds(k * L, L)], 0)
      tbs[pl.ds(k * L, L)] = jnp.where(sel, tbs[pl.ds(k * L, L)], ACC_TRASH)
    soff = w * (EROWS * 128) + srow * 128
    pltpu.sync_copy(tbg.at[pl.ds(0, 128)], sp_s.at[pl.ds(soff, 128)])
    pltpu.sync_copy(tbs.at[pl.ds(0, 128)], sp_d.at[pl.ds(soff, 128)])

  nsp = srow + jnp.where(cnt1 > 0, 1, 0)
  _finalize(gbase0, n16_0)

  _init_acc(gbase1, n16_1)
  wbase = w * (EROWS * 128)
  nblk = nsp // 4
  nrem = nsp - nblk * 4

  def _replay_blk(ib, _):
    pltpu.sync_copy(sp_s.at[pl.ds(wbase + ib * 512, 512)], rbg)
    pltpu.sync_copy(sp_d.at[pl.ds(wbase + ib * 512, 512)], rbs)

    def _rbatch(j, _):
      _sync_batch(jnp.int32(BATCH), j * BATCH, rbg, rbs)
      return 0

    lax.fori_loop(0, 512 // BATCH, _rbatch, 0)
    return 0

  lax.fori_loop(0, nblk, _replay_blk, 0)

  def _replay_row(ir, _):
    roff = wbase + nblk * 512 + ir * 128
    pltpu.sync_copy(sp_s.at[pl.ds(roff, 128)], rbg.at[pl.ds(0, 128)])
    pltpu.sync_copy(sp_d.at[pl.ds(roff, 128)], rbs.at[pl.ds(0, 128)])
    _sync_batch(jnp.int32(BATCH), 0, rbg, rbs)
    _sync_batch(jnp.int32(BATCH), BATCH, rbg, rbs)
    return 0

  lax.fori_loop(0, nrem, _replay_row, 0)
  _finalize(gbase1, n16_1)


@functools.partial(
    pl.kernel,
    out_type=jax.ShapeDtypeStruct((N, D), jnp.float32),
    mesh=plsc.VectorSubcoreMesh(**_MESH),
    compiler_params=pltpu.CompilerParams(needs_layout_passes=False),
    scratch_types=[
        pltpu.VMEM((RPW + 1, D), jnp.float32),
        pltpu.VMEM((16, 128), jnp.int32),
        pltpu.VMEM((16, 128), jnp.int32),
        pltpu.VMEM((SBUF,), jnp.int32),
        pltpu.VMEM((SBUF,), jnp.int32),
        pltpu.VMEM((TBUF,), jnp.int32),
        pltpu.VMEM((TBUF,), jnp.int32),
        pltpu.VMEM((1, BATCH), jnp.int32),
        pltpu.VMEM((BATCH, D), jnp.float32),
        pltpu.VMEM((D,), jnp.float32),
        pltpu.VMEM((16,), jnp.float32),
        pltpu.VMEM((512,), jnp.int32),
        pltpu.VMEM((512,), jnp.int32),
        pltpu.VMEM((16, 128), jnp.int32),
        pltpu.VMEM((16, 128), jnp.int32),
        pltpu.VMEM((BATCH,), jnp.int32),
        pltpu.SemaphoreType.DMA,
        pltpu.SemaphoreType.DMA,
        pltpu.SemaphoreType.DMA,
        pltpu.MemorySpace.HBM((NW * EROWS * 128,), jnp.int32),
        pltpu.MemorySpace.HBM((NW * EROWS * 128,), jnp.int32),
    ],
)
def _agg_kernel(e_ref, g_ref, dinv_ref, b_ref, out_ref,
                acc, srcb, dstb, sbg, sbs, tbg, tbs, gidx, rows_v, bb, dvv,
                rbg, rbs, srcb2, dstb2, pdl, gsem, esem0, esem1, sp_s, sp_d):
  _agg_body(e_ref, g_ref, dinv_ref, b_ref, out_ref,
            acc, srcb, dstb, sbg, sbs, tbg, tbs, gidx, rows_v, bb, dvv,
            rbg, rbs, srcb2, dstb2, pdl, gsem, esem0, esem1, sp_s, sp_d)


def kernel(x, W, b, edge_index):
  npad_e = EROWS * 128 - E
  pad = jnp.stack([
      jnp.zeros((npad_e,), edge_index.dtype),
      jnp.full((npad_e,), NPAD - 1, edge_index.dtype),
  ])
  e3 = jnp.concatenate([edge_index, pad], axis=1).reshape(2, EROWS, 128)
  dinv = _deg_kernel(e3)
  g = _matmul(x, W, dinv[:N].reshape(N, 1))
  return _agg_kernel(e3, g, dinv, b)
